# baseline (speedup 1.0000x reference)
import functools

import jax
import jax.numpy as jnp
from jax import lax
from jax.experimental import pallas as pl
from jax.experimental.pallas import tpu as pltpu
from jax.experimental.pallas import tpu_sc as plsc

_NUM_CHUNKS = 64
_LANES = 16


def _mask_kernel_body(E, W, B, P, NV,
                      base_hbm, rows_hbm, out_hbm,
                      rows_v, buf_a, buf_b,
                      saved_a, saved_b, sidx_a, sidx_b,
                      sem_a, sem_b):
    nvl = NV * _LANES
    wid = lax.axis_index("s") * 2 + lax.axis_index("c")
    off_a = pl.multiple_of(wid * 2 * W, 8)
    off_b = pl.multiple_of((wid * 2 + 1) * W, 8)

    cp_a = pltpu.async_copy(base_hbm.at[pl.ds(off_a, W)], buf_a.at[pl.ds(0, W)], sem_a)
    cp_b = pltpu.async_copy(base_hbm.at[pl.ds(off_b, W)], buf_b.at[pl.ds(0, W)], sem_b)
    pltpu.sync_copy(rows_hbm, rows_v)
    cp_a.wait()
    cp_b.wait()

    zeros_f = jnp.zeros((_LANES,), jnp.float32)

    def scatter_zeros(buf, saved_ref, sidx_ref, b, chunk_off):
        for r in range(P):
            for d in range(NV):
                pos = rows_v[pl.ds((b * P + r) * nvl + d * _LANES, _LANES)]
                local = pos - chunk_off
                ok = (pos >= 0) & (local >= 0) & (local < W)
                sidx = jnp.where(ok, local, W)
                saved = plsc.load_gather(buf, [sidx])
                k = (r * NV + d) * _LANES
                saved_ref[pl.ds(k, _LANES)] = saved
                sidx_ref[pl.ds(k, _LANES)] = sidx
                plsc.store_scatter(buf, [sidx], zeros_f)

    def restore(buf, saved_ref, sidx_ref):
        for k in reversed(range(P * NV)):
            sidx = sidx_ref[pl.ds(k * _LANES, _LANES)]
            saved = saved_ref[pl.ds(k * _LANES, _LANES)]
            plsc.store_scatter(buf, [sidx], saved)

    def fire(buf, sem, b, chunk_off):
        dst = pl.multiple_of(b * E + chunk_off, 8)
        pltpu.async_copy(buf.at[pl.ds(0, W)], out_hbm.at[pl.ds(dst, W)], sem)

    def wait_out(buf, sem):
        pltpu.make_async_copy(buf.at[pl.ds(0, W)], out_hbm.at[pl.ds(0, W)], sem).wait()

    scatter_zeros(buf_a, saved_a, sidx_a, 0, off_a)
    fire(buf_a, sem_a, 0, off_a)
    scatter_zeros(buf_b, saved_b, sidx_b, 0, off_b)
    fire(buf_b, sem_b, 0, off_b)

    def body(b, carry):
        wait_out(buf_a, sem_a)
        restore(buf_a, saved_a, sidx_a)
        scatter_zeros(buf_a, saved_a, sidx_a, b, off_a)
        fire(buf_a, sem_a, b, off_a)
        wait_out(buf_b, sem_b)
        restore(buf_b, saved_b, sidx_b)
        scatter_zeros(buf_b, saved_b, sidx_b, b, off_b)
        fire(buf_b, sem_b, b, off_b)
        return carry

    lax.fori_loop(1, B, body, 0)
    wait_out(buf_a, sem_a)
    wait_out(buf_b, sem_b)


def kernel(base_mask, pert_indices, incidence, incidence_mask):
    del incidence_mask
    E = base_mask.shape[0]
    B, P = pert_indices.shape
    MD = incidence.shape[1]
    assert E % (_NUM_CHUNKS * 8) == 0
    W = E // _NUM_CHUNKS
    NV = -(-MD // _LANES)
    nvl = NV * _LANES

    pert_flat = pert_indices.reshape(-1)
    if pert_flat.dtype != jnp.int32:
        pert_flat = pert_flat.astype(jnp.int32)
    inc = incidence if incidence.dtype == jnp.int32 else incidence.astype(jnp.int32)
    rows = jnp.take(inc, pert_flat, axis=0)
    rows = jnp.pad(rows, ((0, 0), (0, nvl - MD)), constant_values=-1)
    rows_flat = rows.reshape(-1)

    mesh = plsc.VectorSubcoreMesh(core_axis_name="c", subcore_axis_name="s")
    body = functools.partial(_mask_kernel_body, E, W, B, P, NV)
    call = pl.kernel(
        body,
        out_type=jax.ShapeDtypeStruct((B * E,), jnp.float32),
        mesh=mesh,
        compiler_params=pltpu.CompilerParams(needs_layout_passes=False),
        scratch_types=[
            pltpu.VMEM((B * P * nvl,), jnp.int32),
            pltpu.VMEM((W + 8,), jnp.float32),
            pltpu.VMEM((W + 8,), jnp.float32),
            pltpu.VMEM((P * NV * _LANES,), jnp.float32),
            pltpu.VMEM((P * NV * _LANES,), jnp.float32),
            pltpu.VMEM((P * NV * _LANES,), jnp.int32),
            pltpu.VMEM((P * NV * _LANES,), jnp.int32),
            pltpu.SemaphoreType.DMA,
            pltpu.SemaphoreType.DMA,
        ],
    )
    return call(base_mask, rows_flat)
